# Initial kernel scaffold; baseline (speedup 1.0000x reference)
import jax
import jax.numpy as jnp
from jax.experimental import pallas as pl


def kernel(x, W1, b1, W2, b2, w_gate, We1, be1, We2, be2, Wt1, bt1, Wt2, bt2):
    raise NotImplementedError("write your pallas kernel here")


# speedup vs baseline: 8.0802x; 8.0802x over previous
import jax
import jax.numpy as jnp
from jax import lax
from jax.experimental import pallas as pl
from jax.experimental.pallas import tpu as pltpu

_N_TASKS = 3
_N_EXPERTS = 8
_NEG = -1e30


def _pre_gate_body(x_ref, w1_ref, b1_ref, w2_ref, b2_ref, wg_ref,
                   h_ref, gates_ref, aux_ref, *, n_cand):
    X = x_ref[...]
    h1 = jnp.maximum(
        jnp.dot(X, w1_ref[...], preferred_element_type=jnp.float32)
        + b1_ref[...], 0.0)
    h = (jnp.dot(h1, w2_ref[...], preferred_element_type=jnp.float32)
         + b2_ref[...])
    h_ref[...] = h.astype(jnp.bfloat16)
    wg = wg_ref[...]
    n_tok = X.shape[0]
    batch = n_tok // n_cand
    iota = lax.broadcasted_iota(jnp.int32, (n_tok, _N_EXPERTS), 1)
    aux = jnp.zeros((), jnp.float32)
    for i in range(_N_TASKS):
        logits = jnp.dot(h, wg[i], preferred_element_type=jnp.float32)
        m1 = jnp.max(logits, axis=-1, keepdims=True)
        i1 = jnp.min(jnp.where(logits == m1, iota, _N_EXPERTS),
                     axis=-1, keepdims=True)
        mask1 = iota == i1
        l2 = jnp.where(mask1, _NEG, logits)
        m2 = jnp.max(l2, axis=-1, keepdims=True)
        i2 = jnp.min(jnp.where(l2 == m2, iota, _N_EXPERTS),
                     axis=-1, keepdims=True)
        mask2 = iota == i2
        p1 = jax.nn.sigmoid(m1 - m2)
        p2 = jax.nn.sigmoid(m2 - m1)
        g = jnp.where(mask1, p1, jnp.where(mask2, p2, 0.0))
        gates_ref[i] = g
        g3 = g.reshape(batch, n_cand, _N_EXPERTS)
        imp = jnp.sum(g3, axis=0)
        ld = jnp.sum((g3 > 0.0).astype(jnp.float32), axis=0)
        for v in (imp, ld):
            mu = jnp.mean(v, axis=-1, keepdims=True)
            var = jnp.sum((v - mu) ** 2, axis=-1, keepdims=True) / (
                _N_EXPERTS - 1)
            aux = aux + jnp.sum(var / (mu * mu + 1e-10))
    aux_ref[...] = jnp.broadcast_to(0.01 * aux, (1, 1))


_N_CHUNKS = 2


def _expert_tower_body(h_ref, gates_ref, we1_ref, be1_ref, we2_ref,
                       be2_ref, wt1_ref, bt1_ref, wt2_ref, bt2_ref,
                       out_ref, y_ref):
    e = pl.program_id(0)
    n_tok = h_ref.shape[0]
    csz = n_tok // _N_CHUNKS
    onehot = lax.broadcasted_iota(jnp.int32, (1, _N_EXPERTS), 1) == e
    for c in range(_N_CHUNKS):
        sl = pl.ds(c * csz, csz)
        eh = jnp.maximum(
            jnp.dot(h_ref[sl, :], we1_ref[0],
                    preferred_element_type=jnp.float32)
            + be1_ref[0], 0.0)
        eo = jnp.dot(eh.astype(jnp.bfloat16), we2_ref[0],
                     preferred_element_type=jnp.float32)
        for i in range(_N_TASKS):
            gcol = jnp.sum(jnp.where(onehot, gates_ref[i, sl, :], 0.0),
                           axis=-1, keepdims=True)
            contrib = (gcol * eo).astype(jnp.bfloat16)

            @pl.when(e == 0)
            def _(i=i, sl=sl, contrib=contrib):
                y_ref[i, sl, :] = contrib

            @pl.when(e > 0)
            def _(i=i, sl=sl, contrib=contrib):
                y_ref[i, sl, :] = y_ref[i, sl, :] + contrib

    @pl.when(e == _N_EXPERTS - 1)
    def _():
        cols = []
        for i in range(_N_TASKS):
            gb = jnp.dot(gates_ref[i], be2_ref[...],
                         preferred_element_type=jnp.float32)
            yb = (y_ref[i] + gb).astype(jnp.bfloat16)
            t1 = jnp.maximum(
                jnp.dot(yb, wt1_ref[i],
                        preferred_element_type=jnp.float32)
                + bt1_ref[i], 0.0)
            t = (jnp.sum(t1 * wt2_ref[i], axis=-1, keepdims=True)
                 + bt2_ref[i])
            cols.append(jax.nn.sigmoid(t))
        out_ref[...] = jnp.concatenate(cols, axis=1)


def kernel(x, W1, b1, W2, b2, w_gate, We1, be1, We2, be2, Wt1, bt1, Wt2,
           bt2, interpret=False):
    batch, n_cand, d_in = x.shape
    n_tok = batch * n_cand
    hidden = W2.shape[1]
    d_exp = We1.shape[2]
    X = x.reshape(n_tok, d_in)

    h, gates, aux = pl.pallas_call(
        lambda *refs: _pre_gate_body(*refs, n_cand=n_cand),
        out_shape=[
            jax.ShapeDtypeStruct((n_tok, hidden), jnp.bfloat16),
            jax.ShapeDtypeStruct((_N_TASKS, n_tok, _N_EXPERTS), jnp.float32),
            jax.ShapeDtypeStruct((1, 1), jnp.float32),
        ],
        compiler_params=pltpu.CompilerParams(
            vmem_limit_bytes=120 * 1024 * 1024),
        interpret=interpret,
    )(X, W1, b1.reshape(1, -1), W2, b2.reshape(1, -1), w_gate)

    scores = pl.pallas_call(
        _expert_tower_body,
        grid=(_N_EXPERTS,),
        in_specs=[
            pl.BlockSpec((n_tok, hidden), lambda e: (0, 0)),
            pl.BlockSpec((_N_TASKS, n_tok, _N_EXPERTS), lambda e: (0, 0, 0)),
            pl.BlockSpec((1, hidden, d_exp), lambda e: (e, 0, 0)),
            pl.BlockSpec((1, 1, d_exp), lambda e: (e, 0, 0)),
            pl.BlockSpec((1, d_exp, hidden), lambda e: (e, 0, 0)),
            pl.BlockSpec((_N_EXPERTS, hidden), lambda e: (0, 0)),
            pl.BlockSpec((_N_TASKS, hidden, hidden), lambda e: (0, 0, 0)),
            pl.BlockSpec((_N_TASKS, 1, hidden), lambda e: (0, 0, 0)),
            pl.BlockSpec((_N_TASKS, 1, hidden), lambda e: (0, 0, 0)),
            pl.BlockSpec((_N_TASKS, 1, 1), lambda e: (0, 0, 0)),
        ],
        out_specs=pl.BlockSpec((n_tok, _N_TASKS), lambda e: (0, 0)),
        out_shape=jax.ShapeDtypeStruct((n_tok, _N_TASKS), jnp.float32),
        scratch_shapes=[
            pltpu.VMEM((_N_TASKS, n_tok, hidden), jnp.bfloat16),
        ],
        compiler_params=pltpu.CompilerParams(
            dimension_semantics=("arbitrary",),
            vmem_limit_bytes=120 * 1024 * 1024),
        interpret=interpret,
    )(h, gates, We1.astype(jnp.bfloat16), be1.reshape(_N_EXPERTS, 1, -1),
      We2.astype(jnp.bfloat16), be2,
      Wt1.astype(jnp.bfloat16), bt1.reshape(_N_TASKS, 1, -1),
      jnp.transpose(Wt2, (0, 2, 1)), bt2.reshape(_N_TASKS, 1, 1))

    return scores.reshape(batch, n_cand, _N_TASKS), aux.reshape(())


# speedup vs baseline: 8.7368x; 1.0813x over previous
import jax
import jax.numpy as jnp
from jax import lax
from jax.experimental import pallas as pl
from jax.experimental.pallas import tpu as pltpu

_N_TASKS = 3
_N_EXPERTS = 8
_NEG = -1e30


def _pre_gate_body(x_ref, w1_ref, b1_ref, w2_ref, b2_ref, wg_ref,
                   h_ref, gates_ref, aux_ref, *, n_cand):
    X = x_ref[...]
    h1 = jnp.maximum(
        jnp.dot(X, w1_ref[...], preferred_element_type=jnp.float32)
        + b1_ref[...], 0.0)
    h = (jnp.dot(h1, w2_ref[...], preferred_element_type=jnp.float32)
         + b2_ref[...])
    h_ref[...] = h.astype(jnp.bfloat16)
    wg = wg_ref[...]
    n_tok = X.shape[0]
    batch = n_tok // n_cand
    iota = lax.broadcasted_iota(jnp.int32, (n_tok, _N_EXPERTS), 1)
    aux = jnp.zeros((), jnp.float32)
    for i in range(_N_TASKS):
        logits = jnp.dot(h, wg[i], preferred_element_type=jnp.float32)
        m1 = jnp.max(logits, axis=-1, keepdims=True)
        i1 = jnp.min(jnp.where(logits == m1, iota, _N_EXPERTS),
                     axis=-1, keepdims=True)
        mask1 = iota == i1
        l2 = jnp.where(mask1, _NEG, logits)
        m2 = jnp.max(l2, axis=-1, keepdims=True)
        i2 = jnp.min(jnp.where(l2 == m2, iota, _N_EXPERTS),
                     axis=-1, keepdims=True)
        mask2 = iota == i2
        p1 = jax.nn.sigmoid(m1 - m2)
        p2 = jax.nn.sigmoid(m2 - m1)
        g = jnp.where(mask1, p1, jnp.where(mask2, p2, 0.0))
        gates_ref[:, _N_EXPERTS * i:_N_EXPERTS * (i + 1)] = g
        g3 = g.reshape(batch, n_cand, _N_EXPERTS)
        imp = jnp.sum(g3, axis=0)
        ld = jnp.sum((g3 > 0.0).astype(jnp.float32), axis=0)
        for v in (imp, ld):
            mu = jnp.mean(v, axis=-1, keepdims=True)
            var = jnp.sum((v - mu) ** 2, axis=-1, keepdims=True) / (
                _N_EXPERTS - 1)
            aux = aux + jnp.sum(var / (mu * mu + 1e-10))
    aux_ref[...] = jnp.broadcast_to(0.01 * aux, (1, 1))


_N_CHUNKS = 4


def _expert_tower_body(h_ref, gates_ref, we1_ref, be1_ref, we2_ref,
                       be2_ref, wt1_ref, bt1_ref, wt2_ref, bt2_ref,
                       out_ref):
    n_tok = h_ref.shape[0]
    csz = n_tok // _N_CHUNKS

    def chunk(c, _):
        sl = pl.ds(c * csz, csz)
        hc = h_ref[sl, :]
        gc = gates_ref[sl, :]
        acc = [None] * _N_TASKS
        for e in range(_N_EXPERTS):
            eh = jnp.maximum(
                jnp.dot(hc, we1_ref[e],
                        preferred_element_type=jnp.float32)
                + be1_ref[e], 0.0)
            eo = jnp.dot(eh.astype(jnp.bfloat16), we2_ref[e],
                         preferred_element_type=jnp.float32)
            for i in range(_N_TASKS):
                contrib = gc[:, _N_EXPERTS * i + e:_N_EXPERTS * i + e + 1] * eo
                acc[i] = contrib if acc[i] is None else acc[i] + contrib
        cols = []
        for i in range(_N_TASKS):
            gb = jnp.dot(gc[:, _N_EXPERTS * i:_N_EXPERTS * (i + 1)],
                         be2_ref[...], preferred_element_type=jnp.float32)
            yb = (acc[i] + gb).astype(jnp.bfloat16)
            t1 = jnp.maximum(
                jnp.dot(yb, wt1_ref[i],
                        preferred_element_type=jnp.float32)
                + bt1_ref[i], 0.0)
            t = (jnp.sum(t1 * wt2_ref[i], axis=-1, keepdims=True)
                 + bt2_ref[i])
            cols.append(jax.nn.sigmoid(t))
        out_ref[sl, :] = jnp.concatenate(cols, axis=1)
        return _

    lax.fori_loop(0, _N_CHUNKS, chunk, None)


def kernel(x, W1, b1, W2, b2, w_gate, We1, be1, We2, be2, Wt1, bt1, Wt2,
           bt2, interpret=False):
    batch, n_cand, d_in = x.shape
    n_tok = batch * n_cand
    hidden = W2.shape[1]
    d_exp = We1.shape[2]
    X = x.reshape(n_tok, d_in)

    h, gates, aux = pl.pallas_call(
        lambda *refs: _pre_gate_body(*refs, n_cand=n_cand),
        out_shape=[
            jax.ShapeDtypeStruct((n_tok, hidden), jnp.bfloat16),
            jax.ShapeDtypeStruct((n_tok, _N_TASKS * _N_EXPERTS),
                                 jnp.float32),
            jax.ShapeDtypeStruct((1, 1), jnp.float32),
        ],
        compiler_params=pltpu.CompilerParams(
            vmem_limit_bytes=120 * 1024 * 1024),
        interpret=interpret,
    )(X, W1, b1.reshape(1, -1), W2, b2.reshape(1, -1), w_gate)

    scores = pl.pallas_call(
        _expert_tower_body,
        out_shape=jax.ShapeDtypeStruct((n_tok, _N_TASKS), jnp.float32),
        compiler_params=pltpu.CompilerParams(
            vmem_limit_bytes=120 * 1024 * 1024),
        interpret=interpret,
    )(h, gates, We1.astype(jnp.bfloat16), be1.reshape(_N_EXPERTS, 1, -1),
      We2.astype(jnp.bfloat16), be2,
      Wt1.astype(jnp.bfloat16), bt1.reshape(_N_TASKS, 1, -1),
      jnp.transpose(Wt2, (0, 2, 1)), bt2.reshape(_N_TASKS, 1, 1))

    return scores.reshape(batch, n_cand, _N_TASKS), aux.reshape(())
